# Initial kernel scaffold; baseline (speedup 1.0000x reference)
import jax
import jax.numpy as jnp
from jax.experimental import pallas as pl


def kernel(pred_point, gt_point, batch_nums, idx, values, batch_size, epoch_nums):
    raise NotImplementedError("write your pallas kernel here")


# speedup vs baseline: 8.1916x; 8.1916x over previous
import functools

import jax
import jax.numpy as jnp
from jax import lax
from jax.experimental import pallas as pl
from jax.experimental.pallas import tpu as pltpu
from jax.experimental.pallas import tpu_sc as plsc

NC = 2
NS = 16
NW = NC * NS
L = 16
CHUNK = 2048
NBINS = 256
INF_BITS = 0x7F800000

_FIRST = 10
_SECOND = 30
_WIN = 5


def _mesh():
    return plsc.VectorSubcoreMesh(core_axis_name="c", subcore_axis_name="s",
                                  num_cores=NC, num_subcores=NS)


def _wid():
    return lax.axis_index("s") * NC + lax.axis_index("c")


def _iota():
    return lax.iota(jnp.int32, L)


def _bcast_i32(x):
    return jnp.zeros((L,), jnp.int32) + x


def _loss_hist_body(nchunks, B, pf, gf, bn, loss_out, hist_out, psum_out,
                    pbuf, gbuf, bnbuf, lbuf, hist, accref):
    wid = _wid()
    iota = _iota()
    i3 = iota * 3
    ones = jnp.ones((L,), jnp.int32)
    hwords = B * NBINS * L

    def zb(j, _):
        hist[pl.ds(j * L, L)] = jnp.zeros((L,), jnp.int32)
        return 0

    lax.fori_loop(0, hwords // L, zb, 0)
    accref[...] = jnp.zeros((L,), jnp.float32)
    base = wid * (nchunks * CHUNK)

    def chunk_body(c, _):
        p0 = base + c * CHUNK
        pltpu.sync_copy(pf.at[pl.ds(p0 * 3, CHUNK * 3)], pbuf)
        pltpu.sync_copy(gf.at[pl.ds(p0 * 3, CHUNK * 3)], gbuf)
        pltpu.sync_copy(bn.at[pl.ds(p0, CHUNK)], bnbuf)

        def inner(i, _):
            b3 = i * (3 * L)
            px = plsc.load_gather(pbuf, [i3 + b3])
            gx = plsc.load_gather(gbuf, [i3 + b3])
            py = plsc.load_gather(pbuf, [i3 + (b3 + 1)])
            gy = plsc.load_gather(gbuf, [i3 + (b3 + 1)])
            pz = plsc.load_gather(pbuf, [i3 + (b3 + 2)])
            gz = plsc.load_gather(gbuf, [i3 + (b3 + 2)])
            loss = jnp.abs(px - gx) + jnp.abs(py - gy) + jnp.abs(pz - gz)
            lbuf[pl.ds(i * L, L)] = loss
            accref[...] = accref[...] + loss
            seg = bnbuf[pl.ds(i * L, L)]
            bits = plsc.bitcast(loss, jnp.int32)
            digit = lax.shift_right_logical(bits, 24)
            addr = (seg * NBINS + digit) * L + iota
            plsc.addupdate_scatter(hist, [addr], ones)
            return 0

        lax.fori_loop(0, CHUNK // L, inner, 0)
        pltpu.sync_copy(lbuf, loss_out.at[pl.ds(p0, CHUNK)])
        return 0

    lax.fori_loop(0, nchunks, chunk_body, 0)
    pltpu.sync_copy(hist, hist_out.at[wid])
    pltpu.sync_copy(accref, psum_out.at[wid])


def _merge_seg(hist_in, buf, acc, seg):
    span = NBINS * L

    def zb(j, _):
        acc[pl.ds(j * L, L)] = jnp.zeros((L,), jnp.int32)
        return 0

    lax.fori_loop(0, span // L, zb, 0)

    def rbody(r, _):
        pltpu.sync_copy(hist_in.at[r, pl.ds(seg * span, span)], buf)

        def abody(j, _):
            acc[pl.ds(j * L, L)] = acc[pl.ds(j * L, L)] + buf[pl.ds(j * L, L)]
            return 0

        lax.fori_loop(0, span // L, abody, 0)
        return 0

    lax.fori_loop(0, NW, rbody, 0)


def _select_from_acc(acc, k):
    def sbody(b, st):
        cum, sel, cumb = st
        cbin = jnp.sum(acc[pl.ds(b * L, L)])
        hit = (sel < 0) & (cum + cbin > k)
        sel = jnp.where(hit, b, sel)
        cumb = jnp.where(hit, cum, cumb)
        return (cum + cbin, sel, cumb)

    _, sel, cumb = lax.fori_loop(
        0, NBINS, sbody, (jnp.int32(0), jnp.int32(-1), jnp.int32(0)))
    return jnp.where(sel < 0, 0, sel), cumb


def _row_scalar(selbuf, seg, lane):
    row = selbuf[pl.ds(seg * L, L)]
    return jnp.sum(jnp.where(_iota() == lane, row, 0))


def _select_body(first, invN, hist_in, selprev, psum_in, selinfo, lsum,
                 buf, acc, selbuf, rowbuf, psbuf, lrow):
    c = lax.axis_index("c")
    s = lax.axis_index("s")
    iota = _iota()

    @pl.when(c == 0)
    def _():
        seg = s
        _merge_seg(hist_in, buf, acc, seg)
        if first:
            def cbody(b, cum):
                return cum + jnp.sum(acc[pl.ds(b * L, L)])

            cnt = lax.fori_loop(0, NBINS, cbody, jnp.int32(0))
            k = jnp.where(cnt > 0, (cnt - 1) // 2, 0)
            pprev = jnp.int32(0)
        else:
            pltpu.sync_copy(selprev, selbuf)
            pprev = _row_scalar(selbuf, seg, 0)
            k = _row_scalar(selbuf, seg, 1)
            cnt = _row_scalar(selbuf, seg, 2)
        sel, cumb = _select_from_acc(acc, k)
        prefix = lax.shift_left(pprev, 8) | sel
        rank = jnp.where(cnt > 0, k - cumb, 0)
        row = jnp.where(iota == 0, prefix,
                        jnp.where(iota == 1, rank,
                                  jnp.where(iota == 2, cnt, 0)))
        rowbuf[...] = row
        pltpu.sync_copy(rowbuf, selinfo.at[seg])

    if first:
        @pl.when((c == 1) & (s == 0))
        def _():
            pltpu.sync_copy(psum_in, psbuf)

            def pb(r, a):
                return a + psbuf[pl.ds(r * L, L)]

            tot = lax.fori_loop(0, NW, pb, jnp.zeros((L,), jnp.float32))
            total = jnp.sum(tot) * invN
            lrow[...] = jnp.zeros((L,), jnp.float32) + total
            pltpu.sync_copy(lrow, lsum)


def _refine_body(nchunks, B, shift_prefix, shift_digit, loss_in, bn, selprev,
                 hist_out, lbuf, bnbuf, selbuf, hist):
    wid = _wid()
    iota = _iota()
    ones = jnp.ones((L,), jnp.int32)
    hwords = B * NBINS * L

    def zb(j, _):
        hist[pl.ds(j * L, L)] = jnp.zeros((L,), jnp.int32)
        return 0

    lax.fori_loop(0, hwords // L, zb, 0)
    pltpu.sync_copy(selprev, selbuf)
    base = wid * (nchunks * CHUNK)

    def chunk_body(c, _):
        p0 = base + c * CHUNK
        pltpu.sync_copy(loss_in.at[pl.ds(p0, CHUNK)], lbuf)
        pltpu.sync_copy(bn.at[pl.ds(p0, CHUNK)], bnbuf)

        def inner(i, _):
            loss = lbuf[pl.ds(i * L, L)]
            seg = bnbuf[pl.ds(i * L, L)]
            bits = plsc.bitcast(loss, jnp.int32)
            selp = plsc.load_gather(selbuf, [seg * L])
            match = lax.shift_right_logical(bits, shift_prefix) == selp
            digit = lax.shift_right_logical(bits, shift_digit) & 0xFF
            addr = (seg * NBINS + digit) * L + iota
            plsc.addupdate_scatter(hist, [addr], ones, mask=match)
            return 0

        lax.fori_loop(0, CHUNK // L, inner, 0)
        return 0

    lax.fori_loop(0, nchunks, chunk_body, 0)
    pltpu.sync_copy(hist, hist_out.at[wid])


def _med_body(hist_in, selprev, med_out, buf, acc, selbuf, medrow):
    c = lax.axis_index("c")
    s = lax.axis_index("s")

    @pl.when(c == 0)
    def _():
        seg = s
        _merge_seg(hist_in, buf, acc, seg)
        pltpu.sync_copy(selprev, selbuf)
        pprev = _row_scalar(selbuf, seg, 0)
        k = _row_scalar(selbuf, seg, 1)
        cnt = _row_scalar(selbuf, seg, 2)
        sel, _ = _select_from_acc(acc, k)
        med_bits = lax.shift_left(pprev, 16) | lax.shift_left(sel, 8) | 0x80
        med_bits = jnp.where(cnt > 0, med_bits, INF_BITS)
        medrow[...] = plsc.bitcast(_bcast_i32(med_bits), jnp.float32)
        pltpu.sync_copy(medrow, med_out.at[seg])


def _copy_body(vch, ncols, med_in, vin, idx_in, win_in, vout,
               medbuf, copybuf, idxbuf, winbuf):
    wid = _wid()
    iota = _iota()
    pltpu.sync_copy(med_in, medbuf)
    med_vec = plsc.load_gather(medbuf, [iota, jnp.zeros((L,), jnp.int32)])
    lo = wid * vch
    pltpu.sync_copy(vin.at[pl.ds(lo, vch)], copybuf)
    pltpu.sync_copy(idx_in, idxbuf)
    pltpu.sync_copy(win_in, winbuf)
    idxv = idxbuf[...]
    dwv = winbuf[pl.ds(0, L)]
    cev = winbuf[pl.ds(L, L)]
    t = idxv * ncols + cev
    m = (dwv > 0) & (t >= lo) & (t < lo + vch)
    tr = jnp.where(m, t - lo, 0)
    plsc.store_scatter(copybuf, [tr], med_vec, mask=m)
    pltpu.sync_copy(copybuf, vout.at[pl.ds(lo, vch)])


def kernel(pred_point, gt_point, batch_nums, idx, values, batch_size,
           epoch_nums):
    N = pred_point.shape[0]
    B = idx.shape[0]
    assert B == NS and N % (NW * CHUNK) == 0
    nchunks = N // (NW * CHUNK)
    hshape = (NW, B * NBINS * L)

    pf = pred_point.reshape(-1)
    gf = gt_point.reshape(-1)
    bn = batch_nums.astype(jnp.int32)
    idxi = idx.astype(jnp.int32)

    e = jnp.asarray(epoch_nums, jnp.int32)
    in1 = (e >= _FIRST) & (e < _FIRST + _WIN)
    in2 = (e >= _SECOND) & (e < _SECOND + _WIN)
    dw = (in1 | in2).astype(jnp.int32)
    ce = jnp.where(in1, e - _FIRST, jnp.where(in2, e - _SECOND, 0))
    winfo = jnp.concatenate(
        [jnp.full((L,), dw, jnp.int32), jnp.full((L,), ce, jnp.int32)])

    vsize = values.size
    vpad = -(-vsize // (NW * 8)) * (NW * 8)
    vch = vpad // NW
    vflat = jnp.concatenate(
        [values.reshape(-1), jnp.zeros((vpad - vsize,), jnp.float32)])

    span = NBINS * L

    k1 = pl.kernel(
        functools.partial(_loss_hist_body, nchunks, B),
        out_type=(
            jax.ShapeDtypeStruct((N,), jnp.float32),
            jax.ShapeDtypeStruct(hshape, jnp.int32),
            jax.ShapeDtypeStruct((NW, L), jnp.float32),
        ),
        mesh=_mesh(),
        compiler_params=pltpu.CompilerParams(needs_layout_passes=False),
        scratch_types=[
            pltpu.VMEM((CHUNK * 3,), jnp.float32),
            pltpu.VMEM((CHUNK * 3,), jnp.float32),
            pltpu.VMEM((CHUNK,), jnp.int32),
            pltpu.VMEM((CHUNK,), jnp.float32),
            pltpu.VMEM((B * span,), jnp.int32),
            pltpu.VMEM((L,), jnp.float32),
        ],
    )
    loss_map, hist1, psums = k1(pf, gf, bn)

    def make_select(first):
        return pl.kernel(
            functools.partial(_select_body, first, jnp.float32(1.0 / N)),
            out_type=(
                jax.ShapeDtypeStruct((B, L), jnp.int32),
                jax.ShapeDtypeStruct((L,), jnp.float32),
            ),
            mesh=_mesh(),
            compiler_params=pltpu.CompilerParams(needs_layout_passes=False),
            scratch_types=[
                pltpu.VMEM((span,), jnp.int32),
                pltpu.VMEM((span,), jnp.int32),
                pltpu.VMEM((B * L,), jnp.int32),
                pltpu.VMEM((L,), jnp.int32),
                pltpu.VMEM((NW * L,), jnp.float32),
                pltpu.VMEM((L,), jnp.float32),
            ],
        )

    sel1, lsum = make_select(True)(hist1, jnp.zeros((B * L,), jnp.int32),
                                   psums.reshape(-1))

    def make_refine(shift_prefix, shift_digit):
        return pl.kernel(
            functools.partial(_refine_body, nchunks, B, shift_prefix,
                              shift_digit),
            out_type=jax.ShapeDtypeStruct(hshape, jnp.int32),
            mesh=_mesh(),
            compiler_params=pltpu.CompilerParams(needs_layout_passes=False),
            scratch_types=[
                pltpu.VMEM((CHUNK,), jnp.float32),
                pltpu.VMEM((CHUNK,), jnp.int32),
                pltpu.VMEM((B * L,), jnp.int32),
                pltpu.VMEM((B * span,), jnp.int32),
            ],
        )

    hist2 = make_refine(24, 16)(loss_map, bn, sel1.reshape(-1))
    sel2, _ = make_select(False)(hist2, sel1.reshape(-1),
                                 psums.reshape(-1))
    hist3 = make_refine(16, 8)(loss_map, bn, sel2.reshape(-1))

    k6a = pl.kernel(
        _med_body,
        out_type=jax.ShapeDtypeStruct((B, L), jnp.float32),
        mesh=_mesh(),
        compiler_params=pltpu.CompilerParams(needs_layout_passes=False),
        scratch_types=[
            pltpu.VMEM((span,), jnp.int32),
            pltpu.VMEM((span,), jnp.int32),
            pltpu.VMEM((B * L,), jnp.int32),
            pltpu.VMEM((L,), jnp.float32),
        ],
    )
    meds = k6a(hist3, sel2.reshape(-1))

    k6b = pl.kernel(
        functools.partial(_copy_body, vch, values.shape[-1]),
        out_type=jax.ShapeDtypeStruct((vpad,), jnp.float32),
        mesh=_mesh(),
        compiler_params=pltpu.CompilerParams(needs_layout_passes=False),
        scratch_types=[
            pltpu.VMEM((B, L), jnp.float32),
            pltpu.VMEM((vch,), jnp.float32),
            pltpu.VMEM((L,), jnp.int32),
            pltpu.VMEM((2 * L,), jnp.int32),
        ],
    )
    vout = k6b(meds, vflat, idxi, winfo)

    loss = lsum[0]
    values_out = vout[:vsize].reshape(values.shape)
    return (loss, values_out)


# speedup vs baseline: 8.4372x; 1.0300x over previous
import functools

import jax
import jax.numpy as jnp
from jax import lax
from jax.experimental import pallas as pl
from jax.experimental.pallas import tpu as pltpu
from jax.experimental.pallas import tpu_sc as plsc

NC = 2
NS = 16
NW = NC * NS
L = 16
CHUNK = 2048
NBINS = 256
INF_BITS = 0x7F800000

_FIRST = 10
_SECOND = 30
_WIN = 5


def _mesh():
    return plsc.VectorSubcoreMesh(core_axis_name="c", subcore_axis_name="s",
                                  num_cores=NC, num_subcores=NS)


def _wid():
    return lax.axis_index("s") * NC + lax.axis_index("c")


def _iota():
    return lax.iota(jnp.int32, L)


def _bcast_i32(x):
    return jnp.zeros((L,), jnp.int32) + x


def _loss_hist_body(nchunks, B, pf, gf, bn, loss_out, hist_out, psum_out,
                    pbuf, gbuf, bnbuf, lbuf, hist, accref):
    wid = _wid()
    iota = _iota()
    i3 = iota * 3
    ones = jnp.ones((L,), jnp.int32)
    hwords = B * NBINS * L

    def zb(j8, _):
        for u in range(8):
            hist[pl.ds((j8 * 8 + u) * L, L)] = jnp.zeros((L,), jnp.int32)
        return 0

    lax.fori_loop(0, hwords // L // 8, zb, 0)
    accref[...] = jnp.zeros((L,), jnp.float32)
    base = wid * (nchunks * CHUNK)

    def chunk_body(c, _):
        p0 = base + c * CHUNK
        pltpu.sync_copy(pf.at[pl.ds(p0 * 3, CHUNK * 3)], pbuf)
        pltpu.sync_copy(gf.at[pl.ds(p0 * 3, CHUNK * 3)], gbuf)
        pltpu.sync_copy(bn.at[pl.ds(p0, CHUNK)], bnbuf)

        def inner(i8, _):
            acc_l = jnp.zeros((L,), jnp.float32)
            for u in range(8):
                i = i8 * 8 + u
                b3 = i * (3 * L)
                px = plsc.load_gather(pbuf, [i3 + b3])
                gx = plsc.load_gather(gbuf, [i3 + b3])
                py = plsc.load_gather(pbuf, [i3 + (b3 + 1)])
                gy = plsc.load_gather(gbuf, [i3 + (b3 + 1)])
                pz = plsc.load_gather(pbuf, [i3 + (b3 + 2)])
                gz = plsc.load_gather(gbuf, [i3 + (b3 + 2)])
                loss = jnp.abs(px - gx) + jnp.abs(py - gy) + jnp.abs(pz - gz)
                lbuf[pl.ds(i * L, L)] = loss
                acc_l = acc_l + loss
                seg = bnbuf[pl.ds(i * L, L)]
                bits = plsc.bitcast(loss, jnp.int32)
                digit = lax.shift_right_logical(bits, 24)
                addr = (seg * NBINS + digit) * L + iota
                plsc.addupdate_scatter(hist, [addr], ones)
            accref[...] = accref[...] + acc_l
            return 0

        lax.fori_loop(0, CHUNK // L // 8, inner, 0)
        pltpu.sync_copy(lbuf, loss_out.at[pl.ds(p0, CHUNK)])
        return 0

    lax.fori_loop(0, nchunks, chunk_body, 0)
    pltpu.sync_copy(hist, hist_out.at[wid])
    pltpu.sync_copy(accref, psum_out.at[wid])


def _merge_seg(hist_in, buf, acc, seg):
    span = NBINS * L

    def zb(j8, _):
        for u in range(8):
            acc[pl.ds((j8 * 8 + u) * L, L)] = jnp.zeros((L,), jnp.int32)
        return 0

    lax.fori_loop(0, span // L // 8, zb, 0)

    def rbody(r, _):
        pltpu.sync_copy(hist_in.at[r, pl.ds(seg * span, span)], buf)

        def abody(j8, _):
            for u in range(8):
                j = j8 * 8 + u
                acc[pl.ds(j * L, L)] = (acc[pl.ds(j * L, L)]
                                        + buf[pl.ds(j * L, L)])
            return 0

        lax.fori_loop(0, span // L // 8, abody, 0)
        return 0

    lax.fori_loop(0, NW, rbody, 0)


def _select_from_acc(acc, k):
    def sbody(b, st):
        cum, sel, cumb = st
        cbin = jnp.sum(acc[pl.ds(b * L, L)])
        hit = (sel < 0) & (cum + cbin > k)
        sel = jnp.where(hit, b, sel)
        cumb = jnp.where(hit, cum, cumb)
        return (cum + cbin, sel, cumb)

    _, sel, cumb = lax.fori_loop(
        0, NBINS, sbody, (jnp.int32(0), jnp.int32(-1), jnp.int32(0)))
    return jnp.where(sel < 0, 0, sel), cumb


def _row_scalar(selbuf, seg, lane):
    row = selbuf[pl.ds(seg * L, L)]
    return jnp.sum(jnp.where(_iota() == lane, row, 0))


def _select_body(first, invN, hist_in, selprev, psum_in, selinfo, lsum,
                 buf, acc, selbuf, rowbuf, psbuf, lrow):
    c = lax.axis_index("c")
    s = lax.axis_index("s")
    iota = _iota()

    @pl.when(c == 0)
    def _():
        seg = s
        _merge_seg(hist_in, buf, acc, seg)
        if first:
            def cbody(b, cum):
                return cum + jnp.sum(acc[pl.ds(b * L, L)])

            cnt = lax.fori_loop(0, NBINS, cbody, jnp.int32(0))
            k = jnp.where(cnt > 0, (cnt - 1) // 2, 0)
            pprev = jnp.int32(0)
        else:
            pltpu.sync_copy(selprev, selbuf)
            pprev = _row_scalar(selbuf, seg, 0)
            k = _row_scalar(selbuf, seg, 1)
            cnt = _row_scalar(selbuf, seg, 2)
        sel, cumb = _select_from_acc(acc, k)
        prefix = lax.shift_left(pprev, 8) | sel
        rank = jnp.where(cnt > 0, k - cumb, 0)
        row = jnp.where(iota == 0, prefix,
                        jnp.where(iota == 1, rank,
                                  jnp.where(iota == 2, cnt, 0)))
        rowbuf[...] = row
        pltpu.sync_copy(rowbuf, selinfo.at[seg])

    if first:
        @pl.when((c == 1) & (s == 0))
        def _():
            pltpu.sync_copy(psum_in, psbuf)

            def pb(r, a):
                return a + psbuf[pl.ds(r * L, L)]

            tot = lax.fori_loop(0, NW, pb, jnp.zeros((L,), jnp.float32))
            total = jnp.sum(tot) * invN
            lrow[...] = jnp.zeros((L,), jnp.float32) + total
            pltpu.sync_copy(lrow, lsum)


def _refine_body(nchunks, B, shift_prefix, shift_digit, loss_in, bn, selprev,
                 hist_out, lbuf, bnbuf, selbuf, hist):
    wid = _wid()
    iota = _iota()
    ones = jnp.ones((L,), jnp.int32)
    hwords = B * NBINS * L

    def zb(j8, _):
        for u in range(8):
            hist[pl.ds((j8 * 8 + u) * L, L)] = jnp.zeros((L,), jnp.int32)
        return 0

    lax.fori_loop(0, hwords // L // 8, zb, 0)
    pltpu.sync_copy(selprev, selbuf)
    base = wid * (nchunks * CHUNK)

    def chunk_body(c, _):
        p0 = base + c * CHUNK
        pltpu.sync_copy(loss_in.at[pl.ds(p0, CHUNK)], lbuf)
        pltpu.sync_copy(bn.at[pl.ds(p0, CHUNK)], bnbuf)

        def inner(i8, _):
            for u in range(8):
                i = i8 * 8 + u
                loss = lbuf[pl.ds(i * L, L)]
                seg = bnbuf[pl.ds(i * L, L)]
                bits = plsc.bitcast(loss, jnp.int32)
                selp = plsc.load_gather(selbuf, [seg * L])
                match = lax.shift_right_logical(bits, shift_prefix) == selp
                digit = lax.shift_right_logical(bits, shift_digit) & 0xFF
                addr = (seg * NBINS + digit) * L + iota
                plsc.addupdate_scatter(hist, [addr], ones, mask=match)
            return 0

        lax.fori_loop(0, CHUNK // L // 8, inner, 0)
        return 0

    lax.fori_loop(0, nchunks, chunk_body, 0)
    pltpu.sync_copy(hist, hist_out.at[wid])


def _med_body(hist_in, selprev, med_out, buf, acc, selbuf, medrow):
    c = lax.axis_index("c")
    s = lax.axis_index("s")

    @pl.when(c == 0)
    def _():
        seg = s
        _merge_seg(hist_in, buf, acc, seg)
        pltpu.sync_copy(selprev, selbuf)
        pprev = _row_scalar(selbuf, seg, 0)
        k = _row_scalar(selbuf, seg, 1)
        cnt = _row_scalar(selbuf, seg, 2)
        sel, _ = _select_from_acc(acc, k)
        med_bits = lax.shift_left(pprev, 16) | lax.shift_left(sel, 8) | 0x80
        med_bits = jnp.where(cnt > 0, med_bits, INF_BITS)
        medrow[...] = plsc.bitcast(_bcast_i32(med_bits), jnp.float32)
        pltpu.sync_copy(medrow, med_out.at[seg])


def _copy_body(vch, ncols, med_in, vin, idx_in, win_in, vout,
               medbuf, copybuf, idxbuf, winbuf):
    wid = _wid()
    iota = _iota()
    pltpu.sync_copy(med_in, medbuf)
    med_vec = plsc.load_gather(medbuf, [iota, jnp.zeros((L,), jnp.int32)])
    lo = wid * vch
    pltpu.sync_copy(vin.at[pl.ds(lo, vch)], copybuf)
    pltpu.sync_copy(idx_in, idxbuf)
    pltpu.sync_copy(win_in, winbuf)
    idxv = idxbuf[...]
    dwv = winbuf[pl.ds(0, L)]
    cev = winbuf[pl.ds(L, L)]
    t = idxv * ncols + cev
    m = (dwv > 0) & (t >= lo) & (t < lo + vch)
    tr = jnp.where(m, t - lo, 0)
    plsc.store_scatter(copybuf, [tr], med_vec, mask=m)
    pltpu.sync_copy(copybuf, vout.at[pl.ds(lo, vch)])


def kernel(pred_point, gt_point, batch_nums, idx, values, batch_size,
           epoch_nums):
    N = pred_point.shape[0]
    B = idx.shape[0]
    assert B == NS and N % (NW * CHUNK) == 0
    nchunks = N // (NW * CHUNK)
    hshape = (NW, B * NBINS * L)

    pf = pred_point.reshape(-1)
    gf = gt_point.reshape(-1)
    bn = batch_nums.astype(jnp.int32)
    idxi = idx.astype(jnp.int32)

    e = jnp.asarray(epoch_nums, jnp.int32)
    in1 = (e >= _FIRST) & (e < _FIRST + _WIN)
    in2 = (e >= _SECOND) & (e < _SECOND + _WIN)
    dw = (in1 | in2).astype(jnp.int32)
    ce = jnp.where(in1, e - _FIRST, jnp.where(in2, e - _SECOND, 0))
    winfo = jnp.concatenate(
        [jnp.full((L,), dw, jnp.int32), jnp.full((L,), ce, jnp.int32)])

    vsize = values.size
    vpad = -(-vsize // (NW * 8)) * (NW * 8)
    vch = vpad // NW
    vflat = jnp.concatenate(
        [values.reshape(-1), jnp.zeros((vpad - vsize,), jnp.float32)])

    span = NBINS * L

    k1 = pl.kernel(
        functools.partial(_loss_hist_body, nchunks, B),
        out_type=(
            jax.ShapeDtypeStruct((N,), jnp.float32),
            jax.ShapeDtypeStruct(hshape, jnp.int32),
            jax.ShapeDtypeStruct((NW, L), jnp.float32),
        ),
        mesh=_mesh(),
        compiler_params=pltpu.CompilerParams(needs_layout_passes=False),
        scratch_types=[
            pltpu.VMEM((CHUNK * 3,), jnp.float32),
            pltpu.VMEM((CHUNK * 3,), jnp.float32),
            pltpu.VMEM((CHUNK,), jnp.int32),
            pltpu.VMEM((CHUNK,), jnp.float32),
            pltpu.VMEM((B * span,), jnp.int32),
            pltpu.VMEM((L,), jnp.float32),
        ],
    )
    loss_map, hist1, psums = k1(pf, gf, bn)

    def make_select(first):
        return pl.kernel(
            functools.partial(_select_body, first, jnp.float32(1.0 / N)),
            out_type=(
                jax.ShapeDtypeStruct((B, L), jnp.int32),
                jax.ShapeDtypeStruct((L,), jnp.float32),
            ),
            mesh=_mesh(),
            compiler_params=pltpu.CompilerParams(needs_layout_passes=False),
            scratch_types=[
                pltpu.VMEM((span,), jnp.int32),
                pltpu.VMEM((span,), jnp.int32),
                pltpu.VMEM((B * L,), jnp.int32),
                pltpu.VMEM((L,), jnp.int32),
                pltpu.VMEM((NW * L,), jnp.float32),
                pltpu.VMEM((L,), jnp.float32),
            ],
        )

    sel1, lsum = make_select(True)(hist1, jnp.zeros((B * L,), jnp.int32),
                                   psums.reshape(-1))

    def make_refine(shift_prefix, shift_digit):
        return pl.kernel(
            functools.partial(_refine_body, nchunks, B, shift_prefix,
                              shift_digit),
            out_type=jax.ShapeDtypeStruct(hshape, jnp.int32),
            mesh=_mesh(),
            compiler_params=pltpu.CompilerParams(needs_layout_passes=False),
            scratch_types=[
                pltpu.VMEM((CHUNK,), jnp.float32),
                pltpu.VMEM((CHUNK,), jnp.int32),
                pltpu.VMEM((B * L,), jnp.int32),
                pltpu.VMEM((B * span,), jnp.int32),
            ],
        )

    hist2 = make_refine(24, 16)(loss_map, bn, sel1.reshape(-1))
    sel2, _ = make_select(False)(hist2, sel1.reshape(-1),
                                 psums.reshape(-1))
    hist3 = make_refine(16, 8)(loss_map, bn, sel2.reshape(-1))

    k6a = pl.kernel(
        _med_body,
        out_type=jax.ShapeDtypeStruct((B, L), jnp.float32),
        mesh=_mesh(),
        compiler_params=pltpu.CompilerParams(needs_layout_passes=False),
        scratch_types=[
            pltpu.VMEM((span,), jnp.int32),
            pltpu.VMEM((span,), jnp.int32),
            pltpu.VMEM((B * L,), jnp.int32),
            pltpu.VMEM((L,), jnp.float32),
        ],
    )
    meds = k6a(hist3, sel2.reshape(-1))

    k6b = pl.kernel(
        functools.partial(_copy_body, vch, values.shape[-1]),
        out_type=jax.ShapeDtypeStruct((vpad,), jnp.float32),
        mesh=_mesh(),
        compiler_params=pltpu.CompilerParams(needs_layout_passes=False),
        scratch_types=[
            pltpu.VMEM((B, L), jnp.float32),
            pltpu.VMEM((vch,), jnp.float32),
            pltpu.VMEM((L,), jnp.int32),
            pltpu.VMEM((2 * L,), jnp.int32),
        ],
    )
    vout = k6b(meds, vflat, idxi, winfo)

    loss = lsum[0]
    values_out = vout[:vsize].reshape(values.shape)
    return (loss, values_out)
